# baseline (speedup 1.0000x reference)
import functools

import jax
import jax.numpy as jnp
from jax import lax
from jax.experimental import pallas as pl
from jax.experimental.pallas import tpu as pltpu, tpu_sc as plsc

N = 10000
E = 320000
H = 128
COND = 256

NC = 2
NS = 16
NW = NC * NS
C = 128
NCHUNK = 78
PER_TILE = NCHUNK * C
LEFT_BASE = NW * PER_TILE
NLEFT = (E - LEFT_BASE) // C
ZROWS = 624
TAIL = N - NS * ZROWS

BN = 1000


def _tc1_body(x_ref, cond_ref, wct_ref, cb_ref, g_ref, b_ref, h_ref):
    xb = x_ref[...]
    mu = jnp.mean(xb, axis=1, keepdims=True)
    var = jnp.mean((xb - mu) ** 2, axis=1, keepdims=True)
    hn = (xb - mu) * lax.rsqrt(var + 1e-5) * g_ref[...] + b_ref[...]
    cs = jnp.dot(cond_ref[...], wct_ref[...], preferred_element_type=jnp.float32)
    h = hn + cs + cb_ref[...]
    h_ref[...] = h * jax.nn.sigmoid(h)


def _tc1(x, cond, cond_W_T, cond_b, ln1_g, ln1_b):
    grid = (N // BN,)
    return pl.pallas_call(
        _tc1_body,
        grid=grid,
        in_specs=[
            pl.BlockSpec((BN, H), lambda i: (i, 0)),
            pl.BlockSpec((BN, COND), lambda i: (i, 0)),
            pl.BlockSpec((COND, H), lambda i: (0, 0)),
            pl.BlockSpec((1, H), lambda i: (0, 0)),
            pl.BlockSpec((1, H), lambda i: (0, 0)),
            pl.BlockSpec((1, H), lambda i: (0, 0)),
        ],
        out_specs=pl.BlockSpec((BN, H), lambda i: (i, 0)),
        out_shape=jax.ShapeDtypeStruct((N, H), jnp.float32),
    )(x, cond, cond_W_T, cond_b, ln1_g, ln1_b)


def _sc_body(h_hbm, rows_hbm, cols_hbm, adj_hbm, zeros_hbm, out_hbm,
             cv, av, rv, gb, acc, gsem0, gsem1, gsem2, ssem, casem, rsem0, rsem1, rsem2):
    c = lax.axis_index("c")
    s = lax.axis_index("s")
    tid = c * NS + s

    zbase = pl.multiple_of(s * ZROWS, 8)
    pltpu.sync_copy(zeros_hbm.at[pl.ds(zbase, ZROWS)],
                    acc.at[pl.ds(zbase, ZROWS)])

    @pl.when(s == NS - 1)
    def _zero_tail():
        pltpu.sync_copy(zeros_hbm.at[pl.ds(NS * ZROWS, TAIL)],
                        acc.at[pl.ds(NS * ZROWS, TAIL)])

    plsc.subcore_barrier()

    ebase = pl.multiple_of(tid * PER_TILE, 8)

    def eoff(i):
        return pl.multiple_of(ebase + i * C, 8)

    def start_ca(i, b):
        pltpu.async_copy(cols_hbm.at[pl.ds(eoff(i), C)], cv.at[b], casem)
        pltpu.async_copy(adj_hbm.at[pl.ds(eoff(i), C)], av.at[b], casem)

    def wait_ca(b):
        pltpu.make_async_copy(cols_hbm.at[pl.ds(0, C)], cv.at[b], casem).wait()
        pltpu.make_async_copy(adj_hbm.at[pl.ds(0, C)], av.at[b], casem).wait()

    gsems = (gsem0, gsem1, gsem2)
    rsems = (rsem0, rsem1, rsem2)

    def start_rows(i, b):
        pltpu.async_copy(rows_hbm.at[pl.ds(eoff(i), C)], rv.at[b], rsems[b])

    def wait_rows(b):
        pltpu.make_async_copy(rows_hbm.at[pl.ds(0, C)], rv.at[b], rsems[b]).wait()

    def start_gather(b):
        pltpu.async_copy(h_hbm.at[cv.at[b]], gb.at[b], gsems[b])

    def wait_gather(b):
        pltpu.make_async_copy(h_hbm.at[pl.ds(0, C)], gb.at[b], gsems[b]).wait()

    def wait_scatter(b):
        pltpu.make_async_copy(zeros_hbm.at[pl.ds(0, C)], gb.at[b], ssem).wait()

    def scale_chunk(b):
        def scale(g, carry2):
            a16 = av[b, pl.ds(g * 16, 16)]
            for j in range(16):
                a = a16[j]
                e = g * 16 + j
                for k in range(H // 16):
                    gb[b, e, pl.ds(k * 16, 16)] = gb[b, e, pl.ds(k * 16, 16)] * a
            return carry2

        lax.fori_loop(0, C // 16, scale, 0, unroll=False)

    pltpu.sync_copy(cols_hbm.at[pl.ds(eoff(0), C)], cv.at[0])
    pltpu.sync_copy(adj_hbm.at[pl.ds(eoff(0), C)], av.at[0])
    pltpu.sync_copy(rows_hbm.at[pl.ds(eoff(0), C)], rv.at[0])
    start_gather(0)
    start_ca(1, 1)
    start_rows(1, 1)
    wait_ca(1)
    start_gather(1)
    start_ca(2, 2)
    start_rows(2, 2)

    def outer(i3, carry):
        for b in range(3):
            i = i3 * 3 + b
            bn = (b + 2) % 3

            wait_gather(b)
            scale_chunk(b)

            @pl.when(i >= 1)
            def _arrive_rows():
                wait_rows(b)

            @pl.when(i >= 1)
            def _free_prev():
                wait_scatter(bn)

            pltpu.async_copy(gb.at[b], acc.at[rv.at[b]], ssem, add=True)

            @pl.when(i + 2 < NCHUNK)
            def _next_gather():
                wait_ca(bn)
                start_gather(bn)

            @pl.when(i + 3 < NCHUNK)
            def _next_ca():
                start_ca(i + 3, b)

            @pl.when(i + 2 < NCHUNK)
            def _next_rows():
                start_rows(i + 2, bn)
        return carry

    lax.fori_loop(0, NCHUNK // 3, outer, 0, unroll=False)
    wait_scatter(0)

    @pl.when(tid < NLEFT)
    def _leftover():
        off = pl.multiple_of(LEFT_BASE + tid * C, 8)
        pltpu.sync_copy(cols_hbm.at[pl.ds(off, C)], cv.at[0])
        pltpu.sync_copy(adj_hbm.at[pl.ds(off, C)], av.at[0])
        pltpu.sync_copy(rows_hbm.at[pl.ds(off, C)], rv.at[0])
        pltpu.async_copy(h_hbm.at[cv.at[0]], gb.at[0], gsem0).wait()
        scale_chunk(0)
        pltpu.sync_copy(gb.at[0], acc.at[rv.at[0]], add=True)

    plsc.subcore_barrier()
    pltpu.sync_copy(acc.at[pl.ds(zbase, ZROWS)],
                    out_hbm.at[c, pl.ds(zbase, ZROWS)])

    @pl.when(s == NS - 1)
    def _write_tail():
        pltpu.sync_copy(acc.at[pl.ds(NS * ZROWS, TAIL)],
                        out_hbm.at[c, pl.ds(NS * ZROWS, TAIL)])


def _sc_segment_sum(h, rows, cols, adj, zeros):
    mesh = plsc.VectorSubcoreMesh(core_axis_name="c", subcore_axis_name="s")
    fn = pl.kernel(
        _sc_body,
        out_type=jax.ShapeDtypeStruct((NC, N, H), jnp.float32),
        mesh=mesh,
        scratch_types=[
            pltpu.VMEM((3, C), jnp.int32),
            pltpu.VMEM((3, C), jnp.float32),
            pltpu.VMEM((3, C), jnp.int32),
            pltpu.VMEM((3, C, H), jnp.float32),
            pltpu.VMEM_SHARED((N, H), jnp.float32),
            pltpu.SemaphoreType.DMA,
            pltpu.SemaphoreType.DMA,
            pltpu.SemaphoreType.DMA,
            pltpu.SemaphoreType.DMA,
            pltpu.SemaphoreType.DMA,
            pltpu.SemaphoreType.DMA,
            pltpu.SemaphoreType.DMA,
            pltpu.SemaphoreType.DMA,
        ],
    )
    return fn(h, rows, cols, adj, zeros)


def _tc3_body(x_ref, z0_ref, z1_ref, wt_ref, lb_ref, g_ref, b_ref,
              gam_ref, bet_ref, out_ref):
    z = z0_ref[...] + z1_ref[...]
    hb = jnp.dot(z, wt_ref[...], preferred_element_type=jnp.float32) + lb_ref[...]
    mu = jnp.mean(hb, axis=1, keepdims=True)
    var = jnp.mean((hb - mu) ** 2, axis=1, keepdims=True)
    hn = (hb - mu) * lax.rsqrt(var + 1e-5) * g_ref[...] + b_ref[...]
    out_ref[...] = x_ref[...] + hn * gam_ref[...] + bet_ref[...]


def _tc3(x, z0, z1, lin_W_T, lin_b, ln2_g, ln2_b, gamma, beta):
    grid = (N // BN,)
    vec = pl.BlockSpec((1, H), lambda i: (0, 0))
    return pl.pallas_call(
        _tc3_body,
        grid=grid,
        in_specs=[
            pl.BlockSpec((BN, H), lambda i: (i, 0)),
            pl.BlockSpec((BN, H), lambda i: (i, 0)),
            pl.BlockSpec((BN, H), lambda i: (i, 0)),
            pl.BlockSpec((H, H), lambda i: (0, 0)),
            vec, vec, vec, vec, vec,
        ],
        out_specs=pl.BlockSpec((BN, H), lambda i: (i, 0)),
        out_shape=jax.ShapeDtypeStruct((N, H), jnp.float32),
    )(x, z0, z1, lin_W_T, lin_b, ln2_g, ln2_b, gamma, beta)


def kernel(x, edge_index, adj_vals, cond, gamma, beta, lin_W, lin_b,
           ln1_g, ln1_b, ln2_g, ln2_b, cond_W, cond_b):
    rows = edge_index[0].astype(jnp.int32)
    cols = edge_index[1].astype(jnp.int32)
    adj = adj_vals.astype(jnp.float32)
    r2 = lambda v: v.reshape(1, H)

    h = _tc1(x, cond, cond_W.T, r2(cond_b), r2(ln1_g), r2(ln1_b))
    zeros = jnp.zeros((N, H), jnp.float32)
    zp = _sc_segment_sum(h, rows, cols, adj, zeros)
    out = _tc3(x, zp[0], zp[1], lin_W.T, r2(lin_b), r2(ln2_g), r2(ln2_b),
               r2(gamma), r2(beta))
    return out


# speedup vs baseline: 1.0295x; 1.0295x over previous
import jax
import jax.numpy as jnp
from jax import lax
from jax.experimental import pallas as pl
from jax.experimental.pallas import tpu as pltpu, tpu_sc as plsc

N = 10000
E = 320000
H = 128
COND = 256

NC = 2
NS = 16
NW = NC * NS
C = 128
NCHUNK = 78
PER_TILE = NCHUNK * C
LEFT_BASE = NW * PER_TILE
NLEFT = (E - LEFT_BASE) // C
ZROWS = 624
TAIL = N - NS * ZROWS

BN = 1000


def _tc1_body(x_ref, cond_ref, wct_ref, cb_ref, g_ref, b_ref, h_ref):
    xb = x_ref[...]
    mu = jnp.mean(xb, axis=1, keepdims=True)
    var = jnp.mean((xb - mu) ** 2, axis=1, keepdims=True)
    hn = (xb - mu) * lax.rsqrt(var + 1e-5) * g_ref[...] + b_ref[...]
    cs = jnp.dot(cond_ref[...], wct_ref[...], preferred_element_type=jnp.float32)
    h = hn + cs + cb_ref[...]
    h_ref[...] = h * jax.nn.sigmoid(h)


def _tc1(x, cond, cond_W_T, cond_b, ln1_g, ln1_b):
    grid = (N // BN,)
    return pl.pallas_call(
        _tc1_body,
        grid=grid,
        in_specs=[
            pl.BlockSpec((BN, H), lambda i: (i, 0)),
            pl.BlockSpec((BN, COND), lambda i: (i, 0)),
            pl.BlockSpec((COND, H), lambda i: (0, 0)),
            pl.BlockSpec((1, H), lambda i: (0, 0)),
            pl.BlockSpec((1, H), lambda i: (0, 0)),
            pl.BlockSpec((1, H), lambda i: (0, 0)),
        ],
        out_specs=pl.BlockSpec((BN, H), lambda i: (i, 0)),
        out_shape=jax.ShapeDtypeStruct((N, H), jnp.float32),
    )(x, cond, cond_W_T, cond_b, ln1_g, ln1_b)


def _sc_body(h_hbm, rows_hbm, cols_hbm, adj_hbm, out_hbm,
             cv, av, rv, gb, acc,
             gsem0, gsem1, gsem2, ssem, casem, rsem0, rsem1, rsem2):
    c = lax.axis_index("c")
    s = lax.axis_index("s")
    tid = c * NS + s

    def zrow(r, carry):
        for k in range(H // 16):
            gb[0, r, pl.ds(k * 16, 16)] = jnp.zeros((16,), jnp.float32)
        return carry

    lax.fori_loop(0, C, zrow, 0, unroll=False)
    zbase = pl.multiple_of(s * ZROWS, 8)
    for j in range(4):
        pltpu.async_copy(gb.at[0], acc.at[pl.ds(zbase + j * C, C)], ssem)
    pltpu.async_copy(gb.at[0, pl.ds(0, ZROWS - 4 * C)],
                     acc.at[pl.ds(zbase + 4 * C, ZROWS - 4 * C)], ssem)

    @pl.when(s == NS - 1)
    def _zero_tail():
        pltpu.async_copy(gb.at[0, pl.ds(0, TAIL)],
                         acc.at[pl.ds(NS * ZROWS, TAIL)], ssem)
        pltpu.make_async_copy(gb.at[0, pl.ds(0, TAIL)],
                              acc.at[pl.ds(NS * ZROWS, TAIL)], ssem).wait()

    for j in range(4):
        pltpu.make_async_copy(gb.at[0], acc.at[pl.ds(zbase, C)], ssem).wait()
    pltpu.make_async_copy(gb.at[0, pl.ds(0, ZROWS - 4 * C)],
                          acc.at[pl.ds(zbase, ZROWS - 4 * C)], ssem).wait()

    plsc.subcore_barrier()

    ebase = pl.multiple_of(tid * PER_TILE, 8)
    gsems = (gsem0, gsem1, gsem2)
    rsems = (rsem0, rsem1, rsem2)

    def eoff(i):
        return pl.multiple_of(ebase + i * C, 8)

    def start_ca(i, b):
        pltpu.async_copy(cols_hbm.at[pl.ds(eoff(i), C)], cv.at[b], casem)
        pltpu.async_copy(adj_hbm.at[pl.ds(eoff(i), C)], av.at[b], casem)

    def wait_ca(b):
        pltpu.make_async_copy(cols_hbm.at[pl.ds(0, C)], cv.at[b], casem).wait()
        pltpu.make_async_copy(adj_hbm.at[pl.ds(0, C)], av.at[b], casem).wait()

    def start_rows(i, b):
        pltpu.async_copy(rows_hbm.at[pl.ds(eoff(i), C)], rv.at[b], rsems[b])

    def wait_rows(b):
        pltpu.make_async_copy(rows_hbm.at[pl.ds(0, C)], rv.at[b], rsems[b]).wait()

    def start_gather(b):
        pltpu.async_copy(h_hbm.at[cv.at[b]], gb.at[b], gsems[b])

    def wait_gather(b):
        pltpu.make_async_copy(h_hbm.at[pl.ds(0, C)], gb.at[b], gsems[b]).wait()

    def wait_scatter(b):
        pltpu.make_async_copy(h_hbm.at[pl.ds(0, C)], gb.at[b], ssem).wait()

    def scale_chunk(b):
        def scale(g, carry2):
            a16 = av[b, pl.ds(g * 16, 16)]
            for j in range(16):
                a = a16[j]
                e = g * 16 + j
                for k in range(H // 16):
                    gb[b, e, pl.ds(k * 16, 16)] = gb[b, e, pl.ds(k * 16, 16)] * a
            return carry2

        lax.fori_loop(0, C // 16, scale, 0, unroll=False)

    pltpu.sync_copy(cols_hbm.at[pl.ds(eoff(0), C)], cv.at[0])
    pltpu.sync_copy(adj_hbm.at[pl.ds(eoff(0), C)], av.at[0])
    pltpu.sync_copy(rows_hbm.at[pl.ds(eoff(0), C)], rv.at[0])
    start_gather(0)
    start_ca(1, 1)
    start_rows(1, 1)
    wait_ca(1)
    start_gather(1)
    start_ca(2, 2)
    start_rows(2, 2)

    def outer(i3, carry):
        for b in range(3):
            i = i3 * 3 + b
            bn = (b + 2) % 3

            wait_gather(b)
            scale_chunk(b)

            @pl.when(i >= 1)
            def _arrive_rows():
                wait_rows(b)

            @pl.when(i >= 1)
            def _free_prev():
                wait_scatter(bn)

            pltpu.async_copy(gb.at[b], acc.at[rv.at[b]], ssem, add=True)

            @pl.when(i + 2 < NCHUNK)
            def _next_gather():
                wait_ca(bn)
                start_gather(bn)

            @pl.when(i + 3 < NCHUNK)
            def _next_ca():
                start_ca(i + 3, b)

            @pl.when(i + 2 < NCHUNK)
            def _next_rows():
                start_rows(i + 2, bn)
        return carry

    lax.fori_loop(0, NCHUNK // 3, outer, 0, unroll=False)
    wait_scatter(0)

    @pl.when(tid < NLEFT)
    def _leftover():
        off = pl.multiple_of(LEFT_BASE + tid * C, 8)
        pltpu.sync_copy(cols_hbm.at[pl.ds(off, C)], cv.at[0])
        pltpu.sync_copy(adj_hbm.at[pl.ds(off, C)], av.at[0])
        pltpu.sync_copy(rows_hbm.at[pl.ds(off, C)], rv.at[0])
        pltpu.async_copy(h_hbm.at[cv.at[0]], gb.at[0], gsem0).wait()
        scale_chunk(0)
        pltpu.sync_copy(gb.at[0], acc.at[rv.at[0]], add=True)

    plsc.subcore_barrier()
    pltpu.sync_copy(acc.at[pl.ds(zbase, ZROWS)],
                    out_hbm.at[c, pl.ds(zbase, ZROWS)])

    @pl.when(s == NS - 1)
    def _write_tail():
        pltpu.sync_copy(acc.at[pl.ds(NS * ZROWS, TAIL)],
                        out_hbm.at[c, pl.ds(NS * ZROWS, TAIL)])


def _sc_segment_sum(h, rows, cols, adj):
    mesh = plsc.VectorSubcoreMesh(core_axis_name="c", subcore_axis_name="s")
    fn = pl.kernel(
        _sc_body,
        out_type=jax.ShapeDtypeStruct((NC, N, H), jnp.float32),
        mesh=mesh,
        scratch_types=[
            pltpu.VMEM((3, C), jnp.int32),
            pltpu.VMEM((3, C), jnp.float32),
            pltpu.VMEM((3, C), jnp.int32),
            pltpu.VMEM((3, C, H), jnp.float32),
            pltpu.VMEM_SHARED((N, H), jnp.float32),
            pltpu.SemaphoreType.DMA,
            pltpu.SemaphoreType.DMA,
            pltpu.SemaphoreType.DMA,
            pltpu.SemaphoreType.DMA,
            pltpu.SemaphoreType.DMA,
            pltpu.SemaphoreType.DMA,
            pltpu.SemaphoreType.DMA,
            pltpu.SemaphoreType.DMA,
        ],
    )
    return fn(h, rows, cols, adj)


def _tc3_body(x_ref, z0_ref, z1_ref, wt_ref, lb_ref, g_ref, b_ref,
              gam_ref, bet_ref, out_ref):
    z = z0_ref[...] + z1_ref[...]
    hb = jnp.dot(z, wt_ref[...], preferred_element_type=jnp.float32) + lb_ref[...]
    mu = jnp.mean(hb, axis=1, keepdims=True)
    var = jnp.mean((hb - mu) ** 2, axis=1, keepdims=True)
    hn = (hb - mu) * lax.rsqrt(var + 1e-5) * g_ref[...] + b_ref[...]
    out_ref[...] = x_ref[...] + hn * gam_ref[...] + bet_ref[...]


def _tc3(x, z0, z1, lin_W_T, lin_b, ln2_g, ln2_b, gamma, beta):
    grid = (N // BN,)
    vec = pl.BlockSpec((1, H), lambda i: (0, 0))
    return pl.pallas_call(
        _tc3_body,
        grid=grid,
        in_specs=[
            pl.BlockSpec((BN, H), lambda i: (i, 0)),
            pl.BlockSpec((BN, H), lambda i: (i, 0)),
            pl.BlockSpec((BN, H), lambda i: (i, 0)),
            pl.BlockSpec((H, H), lambda i: (0, 0)),
            vec, vec, vec, vec, vec,
        ],
        out_specs=pl.BlockSpec((BN, H), lambda i: (i, 0)),
        out_shape=jax.ShapeDtypeStruct((N, H), jnp.float32),
    )(x, z0, z1, lin_W_T, lin_b, ln2_g, ln2_b, gamma, beta)


def kernel(x, edge_index, adj_vals, cond, gamma, beta, lin_W, lin_b,
           ln1_g, ln1_b, ln2_g, ln2_b, cond_W, cond_b):
    rows = edge_index[0].astype(jnp.int32)
    cols = edge_index[1].astype(jnp.int32)
    adj = adj_vals.astype(jnp.float32)
    r2 = lambda v: v.reshape(1, H)

    h = _tc1(x, cond, cond_W.T, r2(cond_b), r2(ln1_g), r2(ln1_b))
    zp = _sc_segment_sum(h, rows, cols, adj)
    out = _tc3(x, zp[0], zp[1], lin_W.T, r2(lin_b), r2(ln2_g), r2(ln2_b),
               r2(gamma), r2(beta))
    return out
